# baseline (device time: 231352 ns/iter reference)
import jax
import jax.numpy as jnp
from jax import lax
from jax.experimental import pallas as pl
from jax.experimental.pallas import tpu as pltpu

N_DEV = 32
S = 1024
DM = 2048
DKV = 4096
HALF = DKV // 2
H, DH, DR = 16, 128, 32
ROWS = S // N_DEV

_sem_signal = getattr(pl, "semaphore_signal", None) or pltpu.semaphore_signal
_sem_wait = getattr(pl, "semaphore_wait", None) or pltpu.semaphore_wait
_DevIdTy = getattr(pl, "DeviceIdType", None) or pltpu.DeviceIdType
_CompilerParams = getattr(pltpu, "CompilerParams", None) or pltpu.TPUCompilerParams


def _rank_of(d):
    z = d // 8
    k = jnp.mod(d, 8)
    y = k // 2
    m = jnp.mod(k, 4)
    x = jnp.where((m == 1) | (m == 2), 1, 0)
    f = 4 * z + jnp.where(jnp.mod(z, 2) == 0, y, 3 - y)
    return jnp.where(x == 0, f, 31 - f)


def _id_at(r):
    r = jnp.mod(r, N_DEV)
    x = jnp.where(r < 16, 0, 1)
    f = jnp.where(r < 16, r, 31 - r)
    z = f // 4
    yy = jnp.mod(f, 4)
    y = jnp.where(jnp.mod(z, 2) == 0, yy, 3 - yy)
    k = 2 * y + jnp.where(jnp.mod(y, 2) == 0, x, 1 - x)
    return 8 * z + k


def _neighbor_barrier(nbr_a, nbr_b):
    barrier = pltpu.get_barrier_semaphore()
    for nbr in (nbr_a, nbr_b):
        _sem_signal(barrier, inc=1, device_id=(nbr,), device_id_type=_DevIdTy.MESH)
    _sem_wait(barrier, 2)


NS_AR = 4
W_AR = DKV // NS_AR

_FY = (0, 1, 2, 3, 3, 2, 1, 1, 2, 3, 3, 2, 1, 0, 0, 0)
_FZ = (0, 0, 0, 0, 1, 1, 1, 2, 2, 2, 3, 3, 3, 3, 2, 1)
N_F = 16
ROWS_F = S // N_F


def _my_coords(d):
    z = d // 8
    k = jnp.mod(d, 8)
    y = k // 2
    m = jnp.mod(k, 4)
    x = jnp.where((m == 1) | (m == 2), 1, 0)
    return x, y, z


def _id_of(x, y, z):
    k = 2 * y + jnp.where(jnp.mod(y, 2) == 0, x, 1 - x)
    return 8 * z + k


def _frank_of(y, z):
    r = 0
    for i in range(N_F):
        r = r + i * jnp.where((y == _FY[i]) & (z == _FZ[i]), 1, 0)
    return r


def _fid_at(r, x):
    r = jnp.mod(r, N_F)
    y = 0
    z = 0
    for i in range(N_F):
        hit = jnp.where(r == i, 1, 0)
        y = y + _FY[i] * hit
        z = z + _FZ[i] * hit
    return _id_of(x, y, z)


def _pblock_of(r):
    r = jnp.mod(r, N_F)
    c = 0
    for i in range(N_F):
        c = c + (4 * _FZ[i] + _FY[i]) * jnp.where(r == i, 1, 0)
    return c


def _allreduce_body(x_ref, wdkv_ref, wuk_ref, wuv_ref, out_ref,
                    accum_ref, stage_ref, sendbuf_ref, xstage_ref,
                    send_sems, recv_sems, credit_sems, xsend_sems, xrecv_sems):
    d = lax.axis_index("i")
    x, y, z = _my_coords(d)
    frank = _frank_of(y, z)
    nxt = _fid_at(frank + 1, x)
    prv = _fid_at(frank - 1, x)
    mirror = _id_of(1 - x, y, z)

    barrier = pltpu.get_barrier_semaphore()
    for nbr in (nxt, prv, mirror):
        _sem_signal(barrier, inc=1, device_id=(nbr,), device_id_type=_DevIdTy.MESH)

    c = jnp.dot(x_ref[...], wdkv_ref[...],
                preferred_element_type=jnp.float32)
    accum_ref[:, :HALF] = jnp.dot(c, wuk_ref[...],
                                  preferred_element_type=jnp.float32)
    accum_ref[:, HALF:] = jnp.dot(c, wuv_ref[...],
                                  preferred_element_type=jnp.float32)

    _sem_wait(barrier, 3)

    n_rs = N_F - 1

    def mk_rs(st, u):
        ccw = st >= NS_AR // 2
        dst_dev = prv if ccw else nxt
        col0 = st * W_AR
        sgn = -1 if ccw else 1
        slot = u % 2
        send_chunk = jnp.mod(frank - sgn * u, N_F)
        recv_chunk = jnp.mod(frank - sgn * (u + 1), N_F)
        sr = pl.ds(send_chunk * ROWS_F, ROWS_F)
        rr = pl.ds(recv_chunk * ROWS_F, ROWS_F)
        sendbuf_ref[st, slot] = accum_ref[sr, pl.ds(col0, W_AR)].astype(
            jnp.bfloat16)
        send_desc = pltpu.make_async_remote_copy(
            src_ref=sendbuf_ref.at[st, slot],
            dst_ref=stage_ref.at[st, slot],
            send_sem=send_sems.at[st, slot],
            recv_sem=recv_sems.at[st, slot],
            device_id=(dst_dev,),
            device_id_type=_DevIdTy.MESH,
        )
        send_desc.start()
        return send_desc, rr, col0

    n_total = 2 * n_rs

    in_flight = [mk_rs(st, 0) for st in range(NS_AR)]
    for u in range(n_rs):
        slot = u % 2
        for st in range(NS_AR):
            ccw = st >= NS_AR // 2
            ups_dev = nxt if ccw else prv
            send_desc, rr, col0 = in_flight[st]
            send_desc.wait()
            acc = (accum_ref[rr, pl.ds(col0, W_AR)]
                   + stage_ref[st, slot].astype(jnp.float32))
            accum_ref[rr, pl.ds(col0, W_AR)] = acc
            if u <= n_total - 3:
                _sem_signal(credit_sems.at[st, slot], inc=1, device_id=(ups_dev,),
                            device_id_type=_DevIdTy.MESH)
            if u + 1 < n_rs:
                if u + 1 >= 2:
                    _sem_wait(credit_sems.at[st, (u + 1) % 2], 1)
                in_flight[st] = mk_rs(st, u + 1)

    xdescs = []
    for st in range(NS_AR):
        sgn = 1 if st < NS_AR // 2 else -1
        own = jnp.mod(frank + sgn, N_F)
        orr = pl.ds(own * ROWS_F, ROWS_F)
        col0 = st * W_AR
        sendbuf_ref[st, 0] = accum_ref[orr, pl.ds(col0, W_AR)].astype(jnp.bfloat16)
        desc = pltpu.make_async_remote_copy(
            src_ref=sendbuf_ref.at[st, 0],
            dst_ref=xstage_ref.at[st],
            send_sem=xsend_sems.at[st],
            recv_sem=xrecv_sems.at[st],
            device_id=(mirror,),
            device_id_type=_DevIdTy.MESH,
        )
        desc.start()
        xdescs.append((desc, orr, col0))
    for st, (desc, orr, col0) in enumerate(xdescs):
        desc.wait()
        acc = (accum_ref[orr, pl.ds(col0, W_AR)]
               + xstage_ref[st].astype(jnp.float32))
        out_ref[orr, pl.ds(col0, W_AR)] = acc.astype(jnp.bfloat16)

    def mk_ag(st, t):
        ccw = st >= NS_AR // 2
        dst_dev = prv if ccw else nxt
        col0 = st * W_AR
        sgn = -1 if ccw else 1
        slot = (n_rs + t) % 2
        send_chunk = jnp.mod(frank + sgn * (1 - t), N_F)
        recv_chunk = jnp.mod(frank - sgn * t, N_F)
        sr = pl.ds(send_chunk * ROWS_F, ROWS_F)
        rr = pl.ds(recv_chunk * ROWS_F, ROWS_F)
        send_desc = pltpu.make_async_remote_copy(
            src_ref=out_ref.at[sr, pl.ds(col0, W_AR)],
            dst_ref=out_ref.at[sr, pl.ds(col0, W_AR)],
            send_sem=send_sems.at[st, slot],
            recv_sem=recv_sems.at[st, slot],
            device_id=(dst_dev,),
            device_id_type=_DevIdTy.MESH,
        )
        send_desc.start()
        recv_desc = pltpu.make_async_remote_copy(
            src_ref=out_ref.at[rr, pl.ds(col0, W_AR)],
            dst_ref=out_ref.at[rr, pl.ds(col0, W_AR)],
            send_sem=send_sems.at[st, slot],
            recv_sem=recv_sems.at[st, slot],
            device_id=(dst_dev,),
            device_id_type=_DevIdTy.MESH,
        )
        return send_desc, recv_desc

    in_flight = []
    for st in range(NS_AR):
        _sem_wait(credit_sems.at[st, n_rs % 2], 1)
        in_flight.append(mk_ag(st, 0))
    for t in range(n_rs):
        v = n_rs + t
        slot = v % 2
        for st in range(NS_AR):
            ccw = st >= NS_AR // 2
            ups_dev = nxt if ccw else prv
            send_desc, recv_desc = in_flight[st]
            send_desc.wait_send()
            recv_desc.wait_recv()
            if v <= n_total - 3:
                _sem_signal(credit_sems.at[st, slot], inc=1, device_id=(ups_dev,),
                            device_id_type=_DevIdTy.MESH)
            if t + 1 < n_rs:
                _sem_wait(credit_sems.at[st, (v + 1) % 2], 1)
                in_flight[st] = mk_ag(st, t + 1)


def _pallas_allreduce(xm, Wdkv, Wuk, Wuv):
    return pl.pallas_call(
        _allreduce_body,
        out_shape=jax.ShapeDtypeStruct((S, DKV), jnp.bfloat16),
        in_specs=[pl.BlockSpec(memory_space=pltpu.VMEM)] * 4,
        out_specs=pl.BlockSpec(memory_space=pltpu.VMEM),
        scratch_shapes=[
            pltpu.VMEM((S, DKV), jnp.float32),
            pltpu.VMEM((NS_AR, 2, ROWS_F, W_AR), jnp.bfloat16),
            pltpu.VMEM((NS_AR, 2, ROWS_F, W_AR), jnp.bfloat16),
            pltpu.VMEM((NS_AR, ROWS_F, W_AR), jnp.bfloat16),
            pltpu.SemaphoreType.DMA((NS_AR, 2)),
            pltpu.SemaphoreType.DMA((NS_AR, 2)),
            pltpu.SemaphoreType.REGULAR((NS_AR, 2)),
            pltpu.SemaphoreType.DMA((NS_AR,)),
            pltpu.SemaphoreType.DMA((NS_AR,)),
        ],
        compiler_params=_CompilerParams(
            collective_id=0, vmem_limit_bytes=60 * 1024 * 1024),
    )(xm, Wdkv, Wuk, Wuv)


NS_AG = 4
W_AG = DM // NS_AG


def _allgather_body(x_ref, wq_ref, wqr_ref, wkr_ref, wo_ref, kv_ref,
                    out_ref, send_sems, recv_sems, credit_sems):
    d = lax.axis_index("i")
    rank = _rank_of(d)
    nxt = _id_at(rank + 1)
    prv = _id_at(rank - 1)

    barrier = pltpu.get_barrier_semaphore()
    for nbr in (nxt, prv):
        _sem_signal(barrier, inc=1, device_id=(nbr,), device_id_type=_DevIdTy.MESH)

    scale = (DH + DR) ** -0.5
    xq = x_ref[pl.ds(d * ROWS, ROWS), :]
    kr = jnp.dot(x_ref[...], wkr_ref[...],
                 preferred_element_type=jnp.float32)
    q = jnp.dot(xq, wq_ref[...],
                preferred_element_type=jnp.float32)
    qr = jnp.dot(xq, wqr_ref[...],
                 preferred_element_type=jnp.float32)
    cdim = (((1,), (1,)), ((), ()))
    out_acc = jnp.zeros((ROWS, DM), jnp.float32)
    for h in range(H):
        kh = kv_ref[:, pl.ds(h * DH, DH)]
        vh = kv_ref[:, pl.ds(HALF + h * DH, DH)]
        s1 = lax.dot_general(q[:, h * DH:(h + 1) * DH].astype(jnp.bfloat16),
                             kh, cdim, preferred_element_type=jnp.float32)
        s2 = lax.dot_general(qr[:, h * DR:(h + 1) * DR], kr, cdim,
                             preferred_element_type=jnp.float32)
        sc = (s1 + s2) * scale
        m = jnp.max(sc, axis=1, keepdims=True)
        p = jnp.exp(sc - m)
        p = p / jnp.sum(p, axis=1, keepdims=True)
        oh = lax.dot_general(p.astype(jnp.bfloat16), vh,
                             (((1,), (0,)), ((), ())),
                             preferred_element_type=jnp.float32)
        wo_h = wo_ref[pl.ds(h * DH, DH), :].astype(jnp.bfloat16)
        out_acc = out_acc + lax.dot_general(
            oh.astype(jnp.bfloat16), wo_h, (((1,), (0,)), ((), ())),
            preferred_element_type=jnp.float32)

    _sem_wait(barrier, 2)
    out_ref[pl.ds(d * ROWS, ROWS), :] = out_acc.astype(jnp.bfloat16)

    def mk_step(st, t):
        ccw = st >= NS_AG // 2
        dst_dev = prv if ccw else nxt
        col0 = st * W_AG
        sgn = -1 if ccw else 1
        slot = t % 2
        send_chunk = _id_at(rank - sgn * t)
        recv_chunk = _id_at(rank - sgn * (t + 1))
        sr = pl.ds(send_chunk * ROWS, ROWS)
        rr = pl.ds(recv_chunk * ROWS, ROWS)
        send_desc = pltpu.make_async_remote_copy(
            src_ref=out_ref.at[sr, pl.ds(col0, W_AG)],
            dst_ref=out_ref.at[sr, pl.ds(col0, W_AG)],
            send_sem=send_sems.at[st, slot],
            recv_sem=recv_sems.at[st, slot],
            device_id=(dst_dev,),
            device_id_type=_DevIdTy.MESH,
        )
        recv_desc = pltpu.make_async_remote_copy(
            src_ref=out_ref.at[rr, pl.ds(col0, W_AG)],
            dst_ref=out_ref.at[rr, pl.ds(col0, W_AG)],
            send_sem=send_sems.at[st, slot],
            recv_sem=recv_sems.at[st, slot],
            device_id=(dst_dev,),
            device_id_type=_DevIdTy.MESH,
        )
        send_desc.start()
        return send_desc, recv_desc

    n_ag = N_DEV - 1
    in_flight = [mk_step(st, 0) for st in range(NS_AG)]
    for t in range(n_ag):
        slot = t % 2
        for st in range(NS_AG):
            ccw = st >= NS_AG // 2
            ups_dev = nxt if ccw else prv
            send_desc, recv_desc = in_flight[st]
            send_desc.wait_send()
            recv_desc.wait_recv()
            if t <= n_ag - 3:
                _sem_signal(credit_sems.at[st, slot], inc=1, device_id=(ups_dev,),
                            device_id_type=_DevIdTy.MESH)
            if t + 1 < n_ag:
                if t + 1 >= 2:
                    _sem_wait(credit_sems.at[st, (t + 1) % 2], 1)
                in_flight[st] = mk_step(st, t + 1)


def _pallas_attn_allgather(xm, Wq, Wqr, Wkr, Wo, kv):
    return pl.pallas_call(
        _allgather_body,
        out_shape=jax.ShapeDtypeStruct((S, DM), jnp.bfloat16),
        in_specs=[pl.BlockSpec(memory_space=pltpu.VMEM)] * 6,
        out_specs=pl.BlockSpec(memory_space=pltpu.VMEM),
        scratch_shapes=[
            pltpu.SemaphoreType.DMA((NS_AG, 2)),
            pltpu.SemaphoreType.DMA((NS_AG, 2)),
            pltpu.SemaphoreType.REGULAR((NS_AG, 2)),
        ],
        compiler_params=_CompilerParams(
            collective_id=1, vmem_limit_bytes=62 * 1024 * 1024),
    )(xm, Wq, Wqr, Wkr, Wo, kv)


def kernel(x, Wdkv, Wuk, Wuv, Wq, Wqr, Wkr, Wo):
    xm = x[0]
    kv = _pallas_allreduce(xm, Wdkv, Wuk, Wuv)
    out_full = _pallas_attn_allgather(xm, Wq, Wqr, Wkr, Wo, kv)
    return out_full[None].astype(jnp.float32)


# device time: 206879 ns/iter; 1.1183x vs baseline; 1.1183x over previous
import jax
import jax.numpy as jnp
from jax import lax
from jax.experimental import pallas as pl
from jax.experimental.pallas import tpu as pltpu

N_DEV = 32
S = 1024
DM = 2048
DKV = 4096
HALF = DKV // 2
H, DH, DR = 16, 128, 32
ROWS = S // N_DEV

_sem_signal = getattr(pl, "semaphore_signal", None) or pltpu.semaphore_signal
_sem_wait = getattr(pl, "semaphore_wait", None) or pltpu.semaphore_wait
_DevIdTy = getattr(pl, "DeviceIdType", None) or pltpu.DeviceIdType
_CompilerParams = getattr(pltpu, "CompilerParams", None) or pltpu.TPUCompilerParams


def _rank_of(d):
    z = d // 8
    k = jnp.mod(d, 8)
    y = k // 2
    m = jnp.mod(k, 4)
    x = jnp.where((m == 1) | (m == 2), 1, 0)
    f = 4 * z + jnp.where(jnp.mod(z, 2) == 0, y, 3 - y)
    return jnp.where(x == 0, f, 31 - f)


def _id_at(r):
    r = jnp.mod(r, N_DEV)
    x = jnp.where(r < 16, 0, 1)
    f = jnp.where(r < 16, r, 31 - r)
    z = f // 4
    yy = jnp.mod(f, 4)
    y = jnp.where(jnp.mod(z, 2) == 0, yy, 3 - yy)
    k = 2 * y + jnp.where(jnp.mod(y, 2) == 0, x, 1 - x)
    return 8 * z + k


def _neighbor_barrier(nbr_a, nbr_b):
    barrier = pltpu.get_barrier_semaphore()
    for nbr in (nbr_a, nbr_b):
        _sem_signal(barrier, inc=1, device_id=(nbr,), device_id_type=_DevIdTy.MESH)
    _sem_wait(barrier, 2)


NS_AR = 4
W_AR = DKV // NS_AR

_FY = (0, 1, 2, 3, 3, 2, 1, 1, 2, 3, 3, 2, 1, 0, 0, 0)
_FZ = (0, 0, 0, 0, 1, 1, 1, 2, 2, 2, 3, 3, 3, 3, 2, 1)
N_F = 16
ROWS_F = S // N_F


def _my_coords(d):
    z = d // 8
    k = jnp.mod(d, 8)
    y = k // 2
    m = jnp.mod(k, 4)
    x = jnp.where((m == 1) | (m == 2), 1, 0)
    return x, y, z


def _id_of(x, y, z):
    k = 2 * y + jnp.where(jnp.mod(y, 2) == 0, x, 1 - x)
    return 8 * z + k


def _frank_of(y, z):
    r = 0
    for i in range(N_F):
        r = r + i * jnp.where((y == _FY[i]) & (z == _FZ[i]), 1, 0)
    return r


def _fid_at(r, x):
    r = jnp.mod(r, N_F)
    y = 0
    z = 0
    for i in range(N_F):
        hit = jnp.where(r == i, 1, 0)
        y = y + _FY[i] * hit
        z = z + _FZ[i] * hit
    return _id_of(x, y, z)


def _pblock_of(r):
    r = jnp.mod(r, N_F)
    c = 0
    for i in range(N_F):
        c = c + (4 * _FZ[i] + _FY[i]) * jnp.where(r == i, 1, 0)
    return c


def _allreduce_body(x_ref, wdkv_ref, wuk_ref, wuv_ref, out_ref,
                    accum_ref, stage_ref, sendbuf_ref, xstage_ref,
                    send_sems, recv_sems, credit_sems, xsend_sems, xrecv_sems):
    d = lax.axis_index("i")
    x, y, z = _my_coords(d)
    frank = _frank_of(y, z)
    nxt = _fid_at(frank + 1, x)
    prv = _fid_at(frank - 1, x)
    mirror = _id_of(1 - x, y, z)

    barrier = pltpu.get_barrier_semaphore()
    for nbr in (nxt, prv, mirror):
        _sem_signal(barrier, inc=1, device_id=(nbr,), device_id_type=_DevIdTy.MESH)

    c = jnp.dot(x_ref[...], wdkv_ref[...],
                preferred_element_type=jnp.float32)
    accum_ref[:, :HALF] = jnp.dot(c, wuk_ref[...],
                                  preferred_element_type=jnp.float32)
    accum_ref[:, HALF:] = jnp.dot(c, wuv_ref[...],
                                  preferred_element_type=jnp.float32)

    _sem_wait(barrier, 3)

    n_rs = N_F - 1

    def mk_rs(st, u):
        ccw = st >= NS_AR // 2
        dst_dev = prv if ccw else nxt
        col0 = st * W_AR
        sgn = -1 if ccw else 1
        slot = u % 2
        send_chunk = jnp.mod(frank - sgn * u, N_F)
        recv_chunk = jnp.mod(frank - sgn * (u + 1), N_F)
        sr = pl.ds(send_chunk * ROWS_F, ROWS_F)
        rr = pl.ds(recv_chunk * ROWS_F, ROWS_F)
        sendbuf_ref[st, slot] = accum_ref[sr, pl.ds(col0, W_AR)].astype(
            jnp.bfloat16)
        send_desc = pltpu.make_async_remote_copy(
            src_ref=sendbuf_ref.at[st, slot],
            dst_ref=stage_ref.at[st, slot],
            send_sem=send_sems.at[st, slot],
            recv_sem=recv_sems.at[st, slot],
            device_id=(dst_dev,),
            device_id_type=_DevIdTy.MESH,
        )
        send_desc.start()
        return send_desc, rr, col0

    n_total = 2 * n_rs

    in_flight = [mk_rs(st, 0) for st in range(NS_AR)]
    for u in range(n_rs):
        slot = u % 2
        for st in range(NS_AR):
            ccw = st >= NS_AR // 2
            ups_dev = nxt if ccw else prv
            send_desc, rr, col0 = in_flight[st]
            send_desc.wait()
            acc = (accum_ref[rr, pl.ds(col0, W_AR)]
                   + stage_ref[st, slot].astype(jnp.float32))
            accum_ref[rr, pl.ds(col0, W_AR)] = acc
            if u <= n_total - 3:
                _sem_signal(credit_sems.at[st, slot], inc=1, device_id=(ups_dev,),
                            device_id_type=_DevIdTy.MESH)
            if u + 1 < n_rs:
                if u + 1 >= 2:
                    _sem_wait(credit_sems.at[st, (u + 1) % 2], 1)
                in_flight[st] = mk_rs(st, u + 1)

    xdescs = []
    for st in range(NS_AR):
        sgn = 1 if st < NS_AR // 2 else -1
        own = jnp.mod(frank + sgn, N_F)
        orr = pl.ds(own * ROWS_F, ROWS_F)
        col0 = st * W_AR
        sendbuf_ref[st, 0] = accum_ref[orr, pl.ds(col0, W_AR)].astype(jnp.bfloat16)
        desc = pltpu.make_async_remote_copy(
            src_ref=sendbuf_ref.at[st, 0],
            dst_ref=xstage_ref.at[st],
            send_sem=xsend_sems.at[st],
            recv_sem=xrecv_sems.at[st],
            device_id=(mirror,),
            device_id_type=_DevIdTy.MESH,
        )
        desc.start()
        xdescs.append((desc, orr, col0))
    for st, (desc, orr, col0) in enumerate(xdescs):
        desc.wait()
        acc = (accum_ref[orr, pl.ds(col0, W_AR)]
               + xstage_ref[st].astype(jnp.float32))
        out_ref[orr, pl.ds(col0, W_AR)] = acc.astype(jnp.bfloat16)

    def mk_ag(st, t):
        ccw = st >= NS_AR // 2
        dst_dev = prv if ccw else nxt
        col0 = st * W_AR
        sgn = -1 if ccw else 1
        slot = (n_rs + t) % 2
        send_chunk = jnp.mod(frank + sgn * (1 - t), N_F)
        recv_chunk = jnp.mod(frank - sgn * t, N_F)
        sr = pl.ds(send_chunk * ROWS_F, ROWS_F)
        rr = pl.ds(recv_chunk * ROWS_F, ROWS_F)
        send_desc = pltpu.make_async_remote_copy(
            src_ref=out_ref.at[sr, pl.ds(col0, W_AR)],
            dst_ref=out_ref.at[sr, pl.ds(col0, W_AR)],
            send_sem=send_sems.at[st, slot],
            recv_sem=recv_sems.at[st, slot],
            device_id=(dst_dev,),
            device_id_type=_DevIdTy.MESH,
        )
        send_desc.start()
        recv_desc = pltpu.make_async_remote_copy(
            src_ref=out_ref.at[rr, pl.ds(col0, W_AR)],
            dst_ref=out_ref.at[rr, pl.ds(col0, W_AR)],
            send_sem=send_sems.at[st, slot],
            recv_sem=recv_sems.at[st, slot],
            device_id=(dst_dev,),
            device_id_type=_DevIdTy.MESH,
        )
        return send_desc, recv_desc

    in_flight = []
    for st in range(NS_AR):
        _sem_wait(credit_sems.at[st, n_rs % 2], 1)
        in_flight.append(mk_ag(st, 0))
    for t in range(n_rs):
        v = n_rs + t
        slot = v % 2
        for st in range(NS_AR):
            ccw = st >= NS_AR // 2
            ups_dev = nxt if ccw else prv
            send_desc, recv_desc = in_flight[st]
            send_desc.wait_send()
            recv_desc.wait_recv()
            if v <= n_total - 3:
                _sem_signal(credit_sems.at[st, slot], inc=1, device_id=(ups_dev,),
                            device_id_type=_DevIdTy.MESH)
            if t + 1 < n_rs:
                _sem_wait(credit_sems.at[st, (v + 1) % 2], 1)
                in_flight[st] = mk_ag(st, t + 1)


def _pallas_allreduce(xm, Wdkv, Wuk, Wuv):
    return pl.pallas_call(
        _allreduce_body,
        out_shape=jax.ShapeDtypeStruct((S, DKV), jnp.bfloat16),
        in_specs=[pl.BlockSpec(memory_space=pltpu.VMEM)] * 4,
        out_specs=pl.BlockSpec(memory_space=pltpu.VMEM),
        scratch_shapes=[
            pltpu.VMEM((S, DKV), jnp.float32),
            pltpu.VMEM((NS_AR, 2, ROWS_F, W_AR), jnp.bfloat16),
            pltpu.VMEM((NS_AR, 2, ROWS_F, W_AR), jnp.bfloat16),
            pltpu.VMEM((NS_AR, ROWS_F, W_AR), jnp.bfloat16),
            pltpu.SemaphoreType.DMA((NS_AR, 2)),
            pltpu.SemaphoreType.DMA((NS_AR, 2)),
            pltpu.SemaphoreType.REGULAR((NS_AR, 2)),
            pltpu.SemaphoreType.DMA((NS_AR,)),
            pltpu.SemaphoreType.DMA((NS_AR,)),
        ],
        compiler_params=_CompilerParams(
            collective_id=0, vmem_limit_bytes=60 * 1024 * 1024),
    )(xm, Wdkv, Wuk, Wuv)


NS_AG = 4
W_AG = DM // NS_AG


def _allgather_body(x_ref, wq_ref, wqr_ref, wkr_ref, wo_ref, kv_ref,
                    out_ref, send_sems, recv_sems, credit_sems,
                    xsend_sem, xrecv_sem):
    d = lax.axis_index("i")
    x, y, z = _my_coords(d)
    frank = _frank_of(y, z)
    nxt = _fid_at(frank + 1, x)
    prv = _fid_at(frank - 1, x)
    mirror = _id_of(1 - x, y, z)

    barrier = pltpu.get_barrier_semaphore()
    for nbr in (nxt, prv, mirror):
        _sem_signal(barrier, inc=1, device_id=(nbr,), device_id_type=_DevIdTy.MESH)

    scale = (DH + DR) ** -0.5
    xq = x_ref[pl.ds(d * ROWS, ROWS), :]
    kr = jnp.dot(x_ref[...], wkr_ref[...],
                 preferred_element_type=jnp.float32)
    q = jnp.dot(xq, wq_ref[...],
                preferred_element_type=jnp.float32)
    qr = jnp.dot(xq, wqr_ref[...],
                 preferred_element_type=jnp.float32)
    cdim = (((1,), (1,)), ((), ()))
    out_acc = jnp.zeros((ROWS, DM), jnp.float32)
    for h in range(H):
        kh = kv_ref[:, pl.ds(h * DH, DH)]
        vh = kv_ref[:, pl.ds(HALF + h * DH, DH)]
        s1 = lax.dot_general(q[:, h * DH:(h + 1) * DH].astype(jnp.bfloat16),
                             kh, cdim, preferred_element_type=jnp.float32)
        s2 = lax.dot_general(qr[:, h * DR:(h + 1) * DR], kr, cdim,
                             preferred_element_type=jnp.float32)
        sc = (s1 + s2) * scale
        m = jnp.max(sc, axis=1, keepdims=True)
        p = jnp.exp(sc - m)
        p = p / jnp.sum(p, axis=1, keepdims=True)
        oh = lax.dot_general(p.astype(jnp.bfloat16), vh,
                             (((1,), (0,)), ((), ())),
                             preferred_element_type=jnp.float32)
        wo_h = wo_ref[pl.ds(h * DH, DH), :].astype(jnp.bfloat16)
        out_acc = out_acc + lax.dot_general(
            oh.astype(jnp.bfloat16), wo_h, (((1,), (0,)), ((), ())),
            preferred_element_type=jnp.float32)

    _sem_wait(barrier, 3)
    out_ref[pl.ds(d * ROWS, ROWS), :] = out_acc.astype(jnp.bfloat16)

    mr = pl.ds(mirror * ROWS, ROWS)
    xsend = pltpu.make_async_remote_copy(
        src_ref=out_ref.at[pl.ds(d * ROWS, ROWS), :],
        dst_ref=out_ref.at[pl.ds(d * ROWS, ROWS), :],
        send_sem=xsend_sem,
        recv_sem=xrecv_sem,
        device_id=(mirror,),
        device_id_type=_DevIdTy.MESH,
    )
    xsend.start()
    xrecv = pltpu.make_async_remote_copy(
        src_ref=out_ref.at[mr, :],
        dst_ref=out_ref.at[mr, :],
        send_sem=xsend_sem,
        recv_sem=xrecv_sem,
        device_id=(mirror,),
        device_id_type=_DevIdTy.MESH,
    )
    xsend.wait_send()
    xrecv.wait_recv()

    def mk_step(st, t):
        ccw = st >= NS_AG // 2
        dst_dev = prv if ccw else nxt
        col0 = st * W_AG
        sgn = -1 if ccw else 1
        slot = t % 2
        send_blk = _pblock_of(frank - sgn * t)
        recv_blk = _pblock_of(frank - sgn * (t + 1))
        sr = pl.ds(send_blk * ROWS_F, ROWS_F)
        rr = pl.ds(recv_blk * ROWS_F, ROWS_F)
        send_desc = pltpu.make_async_remote_copy(
            src_ref=out_ref.at[sr, pl.ds(col0, W_AG)],
            dst_ref=out_ref.at[sr, pl.ds(col0, W_AG)],
            send_sem=send_sems.at[st, slot],
            recv_sem=recv_sems.at[st, slot],
            device_id=(dst_dev,),
            device_id_type=_DevIdTy.MESH,
        )
        recv_desc = pltpu.make_async_remote_copy(
            src_ref=out_ref.at[rr, pl.ds(col0, W_AG)],
            dst_ref=out_ref.at[rr, pl.ds(col0, W_AG)],
            send_sem=send_sems.at[st, slot],
            recv_sem=recv_sems.at[st, slot],
            device_id=(dst_dev,),
            device_id_type=_DevIdTy.MESH,
        )
        send_desc.start()
        return send_desc, recv_desc

    n_ag = N_F - 1
    in_flight = [mk_step(st, 0) for st in range(NS_AG)]
    for t in range(n_ag):
        slot = t % 2
        for st in range(NS_AG):
            ccw = st >= NS_AG // 2
            ups_dev = nxt if ccw else prv
            send_desc, recv_desc = in_flight[st]
            send_desc.wait_send()
            recv_desc.wait_recv()
            if t <= n_ag - 3:
                _sem_signal(credit_sems.at[st, slot], inc=1, device_id=(ups_dev,),
                            device_id_type=_DevIdTy.MESH)
            if t + 1 < n_ag:
                if t + 1 >= 2:
                    _sem_wait(credit_sems.at[st, (t + 1) % 2], 1)
                in_flight[st] = mk_step(st, t + 1)


def _pallas_attn_allgather(xm, Wq, Wqr, Wkr, Wo, kv):
    return pl.pallas_call(
        _allgather_body,
        out_shape=jax.ShapeDtypeStruct((S, DM), jnp.bfloat16),
        in_specs=[pl.BlockSpec(memory_space=pltpu.VMEM)] * 6,
        out_specs=pl.BlockSpec(memory_space=pltpu.VMEM),
        scratch_shapes=[
            pltpu.SemaphoreType.DMA((NS_AG, 2)),
            pltpu.SemaphoreType.DMA((NS_AG, 2)),
            pltpu.SemaphoreType.REGULAR((NS_AG, 2)),
            pltpu.SemaphoreType.DMA,
            pltpu.SemaphoreType.DMA,
        ],
        compiler_params=_CompilerParams(
            collective_id=1, vmem_limit_bytes=62 * 1024 * 1024),
    )(xm, Wq, Wqr, Wkr, Wo, kv)


def kernel(x, Wdkv, Wuk, Wuv, Wq, Wqr, Wkr, Wo):
    xm = x[0]
    kv = _pallas_allreduce(xm, Wdkv, Wuk, Wuv)
    out_full = _pallas_attn_allgather(xm, Wq, Wqr, Wkr, Wo, kv)
    return out_full[None].astype(jnp.float32)
